# Initial kernel scaffold; baseline (speedup 1.0000x reference)
import jax
import jax.numpy as jnp
from jax.experimental import pallas as pl


def kernel(ego_input, exo_bank, W_t, b_t, W_q, W_k, W_v, W_o, W_cv, b_cv, W_cn, b_cn, k, k_active):
    raise NotImplementedError("write your pallas kernel here")


# speedup vs baseline: 4.0867x; 4.0867x over previous
import functools

import jax
import jax.numpy as jnp
from jax import lax
from jax.experimental import pallas as pl
from jax.experimental.pallas import tpu as pltpu
from jax.experimental.pallas import tpu_sc as plsc

_B = 1024
_D = 64
_N = 100000
_K = 20
_KPAD = 128
_NT = 2048
_T = (_N + _NT - 1) // _NT
_NEG = float("-inf")


def _proj_body(ego_ref, wt_ref, bt_ref, zego_ref, qn_ref):
    z = jnp.dot(ego_ref[...], wt_ref[...], preferred_element_type=jnp.float32)
    z = z + bt_ref[...]
    zego_ref[...] = z
    nrm = jnp.sqrt(jnp.sum(z * z, axis=1, keepdims=True))
    qn_ref[...] = z / (nrm + 1e-6)


def _project(ego, W_t, b_t2):
    return pl.pallas_call(
        _proj_body,
        out_shape=(
            jax.ShapeDtypeStruct((_B, _D), jnp.float32),
            jax.ShapeDtypeStruct((_B, _D), jnp.float32),
        ),
    )(ego, W_t, b_t2)


def _knn_body(qn_ref, bankT_ref, idx_ref, s_ref, vals_ref, gidx_ref, m_ref,
              act_ref):
    t = pl.program_id(0)
    lane = lax.broadcasted_iota(jnp.int32, (_B, _KPAD), 1)

    @pl.when(t == 0)
    def _init():
        vals_ref[...] = jnp.where(lane < _K, _NEG, jnp.inf)
        gidx_ref[...] = jnp.zeros((_B, _KPAD), jnp.int32)

    bt = bankT_ref[...]
    nrm = jnp.sqrt(jnp.sum(bt * bt, axis=0, keepdims=True))
    bn = bt / (nrm + 1e-6)
    s = jnp.dot(qn_ref[...], bn, preferred_element_type=jnp.float32)
    col = lax.broadcasted_iota(jnp.int32, (_B, _NT), 1)
    gcol = col + t * _NT
    s = jnp.where(gcol < _N, s, _NEG)
    s_ref[...] = s
    m0 = jnp.max(s, axis=1, keepdims=True)
    m_ref[...] = m0
    bmin0 = jnp.min(vals_ref[...], axis=1, keepdims=True)
    act_ref[0] = jnp.any(m0 > bmin0).astype(jnp.int32)

    def _step(_, carry):
        @pl.when(act_ref[0] == 1)
        def _merge():
            s_cur = s_ref[...]
            m = m_ref[...]
            am = jnp.min(jnp.where(s_cur == m, col, jnp.int32(2**30)),
                         axis=1, keepdims=True)
            vals = vals_ref[...]
            bmin = jnp.min(vals, axis=1, keepdims=True)
            upd = m > bmin
            bpos = jnp.min(jnp.where(vals == bmin, lane, jnp.int32(2**30)),
                           axis=1, keepdims=True)
            sel = (lane == bpos) & upd
            vals_ref[...] = jnp.where(sel, m, vals)
            gidx_ref[...] = jnp.where(sel, am + t * _NT, gidx_ref[...])
            s_new = jnp.where((col == am) & upd, _NEG, s_cur)
            s_ref[...] = s_new
            m2 = jnp.max(s_new, axis=1, keepdims=True)
            m_ref[...] = m2
            bmin2 = jnp.min(vals_ref[...], axis=1, keepdims=True)
            act_ref[0] = jnp.any(m2 > bmin2).astype(jnp.int32)
        return carry

    lax.fori_loop(0, _K, _step, 0)

    @pl.when(t == _T - 1)
    def _emit():
        idx_ref[...] = gidx_ref[...]


def _knn_topk(qn, bankT):
    return pl.pallas_call(
        _knn_body,
        grid=(_T,),
        in_specs=[
            pl.BlockSpec((_B, _D), lambda t: (0, 0)),
            pl.BlockSpec((_D, _NT), lambda t: (0, t)),
        ],
        out_specs=pl.BlockSpec((_B, _KPAD), lambda t: (0, 0)),
        out_shape=jax.ShapeDtypeStruct((_B, _KPAD), jnp.int32),
        scratch_shapes=[
            pltpu.VMEM((_B, _NT), jnp.float32),
            pltpu.VMEM((_B, _KPAD), jnp.float32),
            pltpu.VMEM((_B, _KPAD), jnp.int32),
            pltpu.VMEM((_B, 1), jnp.float32),
            pltpu.SMEM((1,), jnp.int32),
        ],
    )(qn, bankT)


_NROWS = _B * _K
_NW = 32
_RPW = _NROWS // _NW


def _sc_gather(idx_flat, table):
    mesh = plsc.VectorSubcoreMesh(core_axis_name="c", subcore_axis_name="s")

    @functools.partial(
        pl.kernel,
        mesh=mesh,
        out_type=jax.ShapeDtypeStruct((_NROWS, _D), jnp.float32),
        scratch_types=[
            pltpu.VMEM((_RPW,), jnp.int32),
            pltpu.VMEM((_RPW, _D), jnp.float32),
            pltpu.SemaphoreType.DMA,
        ],
        compiler_params=pltpu.CompilerParams(use_tc_tiling_on_sc=False),
    )
    def _gk(idx_hbm, table_hbm, out_hbm, idx_v, rows_v, sem):
        wid = lax.axis_index("s") * 2 + lax.axis_index("c")
        base = wid * _RPW
        pltpu.sync_copy(idx_hbm.at[pl.ds(base, _RPW)], idx_v)
        pltpu.async_copy(table_hbm.at[idx_v], rows_v, sem).wait()
        pltpu.sync_copy(rows_v, out_hbm.at[pl.ds(base, _RPW)])

    return _gk(idx_flat, table)


_BS = 256


def _attn_body(zego_ref, zexo_ref, wq_ref, wk_ref, wv_ref, wo_ref,
               wcv_ref, bcv_ref, wcn_ref, bcn_ref, mask_ref,
               z_ref, verb_ref, noun_ref):
    z_e = zego_ref[...]
    ze2 = zexo_ref[...]
    q = jnp.dot(z_e, wq_ref[...], preferred_element_type=jnp.float32)
    kk = jnp.dot(ze2, wk_ref[...], preferred_element_type=jnp.float32)
    vv = jnp.dot(ze2, wv_ref[...], preferred_element_type=jnp.float32)
    kk3 = kk.reshape(_BS, _K, _D)
    l = jnp.sum(q[:, None, :] * kk3, axis=2) * (1.0 / 8.0)
    l = l + mask_ref[...]
    l = l - jnp.max(l, axis=1, keepdims=True)
    p = jnp.exp(l)
    p = p / jnp.sum(p, axis=1, keepdims=True)
    vv3 = vv.reshape(_BS, _K, _D)
    ctx = jnp.sum(p[:, :, None] * vv3, axis=1)
    z = z_e + jnp.dot(ctx, wo_ref[...], preferred_element_type=jnp.float32)
    z_ref[...] = z
    verb_ref[...] = jnp.dot(z, wcv_ref[...],
                            preferred_element_type=jnp.float32) + bcv_ref[...]
    noun_ref[...] = jnp.dot(z, wcn_ref[...],
                            preferred_element_type=jnp.float32) + bcn_ref[...]


def _attn_heads(zego, zexo_flat, W_q, W_k, W_v, W_o, W_cv, b_cv2, W_cn,
                b_cn2, maskadd):
    nv = W_cv.shape[1]
    nn = W_cn.shape[1]
    grid = (_B // _BS,)
    wspec = pl.BlockSpec((_D, _D), lambda i: (0, 0))
    return pl.pallas_call(
        _attn_body,
        grid=grid,
        in_specs=[
            pl.BlockSpec((_BS, _D), lambda i: (i, 0)),
            pl.BlockSpec((_BS * _K, _D), lambda i: (i, 0)),
            wspec, wspec, wspec, wspec,
            pl.BlockSpec((_D, nv), lambda i: (0, 0)),
            pl.BlockSpec((1, nv), lambda i: (0, 0)),
            pl.BlockSpec((_D, nn), lambda i: (0, 0)),
            pl.BlockSpec((1, nn), lambda i: (0, 0)),
            pl.BlockSpec((1, _K), lambda i: (0, 0)),
        ],
        out_specs=(
            pl.BlockSpec((_BS, _D), lambda i: (i, 0)),
            pl.BlockSpec((_BS, nv), lambda i: (i, 0)),
            pl.BlockSpec((_BS, nn), lambda i: (i, 0)),
        ),
        out_shape=(
            jax.ShapeDtypeStruct((_B, _D), jnp.float32),
            jax.ShapeDtypeStruct((_B, nv), jnp.float32),
            jax.ShapeDtypeStruct((_B, nn), jnp.float32),
        ),
    )(zego, zexo_flat, W_q, W_k, W_v, W_o, W_cv, b_cv2, W_cn, b_cn2, maskadd)


def kernel(ego_input, exo_bank, W_t, b_t, W_q, W_k, W_v, W_o, W_cv, b_cv,
           W_cn, b_cn, k, k_active):
    zego, qn = _project(ego_input, W_t, b_t.reshape(1, _D))
    bankT = exo_bank.T
    idx_pad = _knn_topk(qn, bankT)
    idx_flat = idx_pad[:, :_K].reshape(_NROWS)
    zexo_flat = _sc_gather(idx_flat, exo_bank)
    maskadd = jnp.where(jnp.arange(_K)[None, :] < k_active,
                        0.0, -1e9).astype(jnp.float32)
    z, verb, noun = _attn_heads(zego, zexo_flat, W_q, W_k, W_v, W_o,
                                W_cv, b_cv.reshape(1, -1),
                                W_cn, b_cn.reshape(1, -1), maskadd)
    return (z, verb, noun)
